# baseline (speedup 1.0000x reference)
import functools

import jax
import jax.numpy as jnp
from jax import lax
from jax.experimental import pallas as pl
from jax.experimental.pallas import tpu as pltpu
from jax.experimental.pallas import tpu_sc as plsc

N = 10000
E = 320000
D = 128
H = 128
ITERS = 3
EPS = 1e-5

NC = 2
NS = 16
NW = NC * NS
LANES = 128

NPAD = 10240
RPT = NPAD // NS
EPAD = 327680
EPW = EPAD // NW
CHUNKS = EPW // LANES

_MESH = plsc.VectorSubcoreMesh(core_axis_name="c", subcore_axis_name="s")


NBUF = 2
GROUPS = CHUNKS // NBUF


@functools.partial(
    pl.kernel,
    out_type=jax.ShapeDtypeStruct((NC, NPAD, D), jnp.float32),
    mesh=_MESH,
    scratch_types=[
        pltpu.VMEM((4, LANES), jnp.int32),
        pltpu.VMEM((4, LANES), jnp.int32),
        pltpu.VMEM((NBUF, LANES, D), jnp.float32),
        pltpu.VMEM_SHARED((NPAD, D), jnp.float32),
        pltpu.SemaphoreType.DMA,
        pltpu.SemaphoreType.DMA,
        pltpu.SemaphoreType.DMA,
    ],
)
def _seg_kernel(table_hbm, src_hbm, dst_hbm, zeros_hbm, out_hbm,
                sidx, didx, rows, acc, gsem, ssem0, ssem1):
    ssems = (ssem0, ssem1)
    cid = lax.axis_index("c")
    sid = lax.axis_index("s")
    rbase = sid * RPT
    pltpu.sync_copy(zeros_hbm.at[pl.ds(rbase, RPT)], acc.at[pl.ds(rbase, RPT)])
    wbase = (cid * NS + sid) * CHUNKS
    plsc.subcore_barrier()

    def load_idx(j, q):
        pltpu.sync_copy(src_hbm.at[pl.ds(wbase + j, 1)], sidx.at[pl.ds(q, 1)])
        pltpu.sync_copy(dst_hbm.at[pl.ds(wbase + j, 1)], didx.at[pl.ds(q, 1)])

    for q in range(3):
        load_idx(q, q)
    pltpu.async_copy(table_hbm.at[sidx.at[0]], rows.at[0], gsem)

    def quad(i, carry):
        for u in range(4):
            j = i * 4 + u
            q = u
            b = u % 2
            nb = 1 - b
            pltpu.make_async_copy(table_hbm.at[sidx.at[q]], rows.at[b],
                                  gsem).wait()
            pltpu.async_copy(rows.at[b], acc.at[didx.at[q]], ssems[b],
                             add=True)
            @pl.when(j >= 1)
            def _():
                pltpu.make_async_copy(rows.at[nb], acc.at[didx.at[(q + 3) % 4]],
                                      ssems[nb]).wait()
            @pl.when(j + 1 < CHUNKS)
            def _():
                pltpu.async_copy(table_hbm.at[sidx.at[(q + 1) % 4]],
                                 rows.at[nb], gsem)
            @pl.when(j + 3 < CHUNKS)
            def _():
                load_idx(j + 3, (q + 3) % 4)
        return carry

    lax.fori_loop(0, CHUNKS // 4, quad, 0)
    pltpu.make_async_copy(rows.at[1], acc.at[didx.at[3]], ssems[1]).wait()
    plsc.subcore_barrier()
    pltpu.sync_copy(acc.at[pl.ds(rbase, RPT)],
                    out_hbm.at[cid, pl.ds(rbase, RPT)])


@functools.partial(
    pl.kernel,
    out_type=jax.ShapeDtypeStruct((NC, NPAD, D), jnp.float32),
    mesh=_MESH,
    scratch_types=[
        pltpu.VMEM((CHUNKS, LANES), jnp.int32),
        pltpu.VMEM((LANES, D), jnp.float32),
        pltpu.VMEM_SHARED((NPAD, D), jnp.float32),
        pltpu.SemaphoreType.DMA,
    ],
)
def _deg_kernel(dst_hbm, ones_hbm, zeros_hbm, out_hbm, didx, ones_v, acc,
                ssem):
    cid = lax.axis_index("c")
    sid = lax.axis_index("s")
    rbase = sid * RPT
    pltpu.sync_copy(zeros_hbm.at[pl.ds(rbase, RPT)], acc.at[pl.ds(rbase, RPT)])
    pltpu.sync_copy(ones_hbm, ones_v)
    wbase = (cid * NS + sid) * CHUNKS
    pltpu.sync_copy(dst_hbm.at[pl.ds(wbase, CHUNKS)], didx)
    plsc.subcore_barrier()

    def fire(i, carry):
        pltpu.async_copy(ones_v, acc.at[didx.at[i]], ssem, add=True)
        return carry

    lax.fori_loop(0, CHUNKS, fire, 0)

    def drain(i, carry):
        pltpu.make_async_copy(ones_v, acc.at[didx.at[0]], ssem).wait()
        return carry

    lax.fori_loop(0, CHUNKS, drain, 0)
    plsc.subcore_barrier()
    pltpu.sync_copy(acc.at[pl.ds(rbase, RPT)],
                    out_hbm.at[cid, pl.ds(rbase, RPT)])


def _leaky(x):
    return jnp.where(x >= 0, x, 0.01 * x)


def _bn(h, g, be):
    mu = jnp.mean(h, axis=0, keepdims=True)
    var = jnp.mean((h - mu) ** 2, axis=0, keepdims=True)
    return g * (h - mu) * lax.rsqrt(var + EPS) + be


def _dinv_col(degp):
    deg = degp[0, :N, 0:1] + degp[1, :N, 0:1] + 1.0
    return lax.rsqrt(deg)


def _stage1_body(x_ref, w_ref, b_ref, g_ref, be_ref, wg_ref, degp_ref,
                 o_ref):
    h = jnp.dot(x_ref[...], w_ref[...],
                preferred_element_type=jnp.float32) + b_ref[...]
    h = _leaky(_bn(h, g_ref[...], be_ref[...]))
    h = h * lax.rsqrt(jnp.sum(h * h))
    dcol = _dinv_col(degp_ref[...])
    o_ref[...] = jnp.dot(h, wg_ref[...],
                         preferred_element_type=jnp.float32) * dcol


def _mid_body(parts_ref, xws_ref, degp_ref, w_ref, b_ref, g_ref, be_ref,
              o_ref):
    dcol = _dinv_col(degp_ref[...])
    parts = parts_ref[...]
    nb = dcol * (parts[0, :N, :] + parts[1, :N, :] + xws_ref[...]) + b_ref[...]
    h2 = _leaky(_bn(nb, g_ref[...], be_ref[...]))
    o_ref[...] = jnp.dot(h2, w_ref[...],
                         preferred_element_type=jnp.float32) * dcol


def _final_body(parts_ref, xws_ref, degp_ref, b_ref, g_ref, be_ref, o_ref):
    dcol = _dinv_col(degp_ref[...])
    parts = parts_ref[...]
    nb = dcol * (parts[0, :N, :] + parts[1, :N, :] + xws_ref[...]) + b_ref[...]
    h2 = _leaky(_bn(nb, g_ref[...], be_ref[...]))
    o_ref[...] = h2 * lax.rsqrt(jnp.sum(h2 * h2))


def _tc(body, *args):
    return pl.pallas_call(
        body, out_shape=jax.ShapeDtypeStruct((N, H), jnp.float32))(*args)


def kernel(x, edge_index, W_in, b_in, g1, be1, W_g, b_g, g2, be2):
    src = jnp.concatenate(
        [edge_index[0], jnp.arange(EPAD - E, dtype=jnp.int32) % N]
    ).reshape(NW * CHUNKS, LANES)
    dst = jnp.concatenate(
        [edge_index[1], jnp.full((EPAD - E,), N, jnp.int32)]
    ).reshape(NW * CHUNKS, LANES)

    zeros128 = jnp.zeros((NPAD, D), jnp.float32)
    ones_blk = jnp.ones((LANES, D), jnp.float32)

    brow = b_in.reshape(1, H)
    g1r = g1.reshape(1, H)
    be1r = be1.reshape(1, H)
    bgr = b_g.reshape(1, H)
    g2r = g2.reshape(1, H)
    be2r = be2.reshape(1, H)

    degp = _deg_kernel(dst, ones_blk, zeros128)

    xws = _tc(_stage1_body, x, W_in, brow, g1r, be1r, W_g, degp)
    for it in range(ITERS):
        parts = _seg_kernel(xws, src, dst, zeros128)
        if it < ITERS - 1:
            xws = _tc(_mid_body, parts, xws, degp, W_g, bgr, g2r, be2r)
        else:
            out = _tc(_final_body, parts, xws, degp, bgr, g2r, be2r)
    return out


# speedup vs baseline: 1.0097x; 1.0097x over previous
import functools

import jax
import jax.numpy as jnp
from jax import lax
from jax.experimental import pallas as pl
from jax.experimental.pallas import tpu as pltpu
from jax.experimental.pallas import tpu_sc as plsc

N = 10000
E = 320000
D = 128
H = 128
ITERS = 3
EPS = 1e-5

NC = 2
NS = 16
NW = NC * NS
LANES = 128

NPAD = 10240
RPT = NPAD // NS
EPAD = 327680
EPW = EPAD // NW
CHUNKS = EPW // LANES

_MESH = plsc.VectorSubcoreMesh(core_axis_name="c", subcore_axis_name="s")


NBUF = 2
GROUPS = CHUNKS // NBUF


@functools.partial(
    pl.kernel,
    out_type=jax.ShapeDtypeStruct((NC, NPAD, D), jnp.float32),
    mesh=_MESH,
    scratch_types=[
        pltpu.VMEM((4, LANES), jnp.int32),
        pltpu.VMEM((4, LANES), jnp.int32),
        pltpu.VMEM((NBUF, LANES, D), jnp.float32),
        pltpu.VMEM_SHARED((NPAD, D), jnp.float32),
        pltpu.SemaphoreType.DMA,
        pltpu.SemaphoreType.DMA,
        pltpu.SemaphoreType.DMA,
        [pltpu.SemaphoreType.DMA] * 4,
    ],
)
def _seg_kernel(table_hbm, src_hbm, dst_hbm, zeros_hbm, out_hbm,
                sidx, didx, rows, acc, gsem, ssem0, ssem1, isems):
    ssems = (ssem0, ssem1)
    cid = lax.axis_index("c")
    sid = lax.axis_index("s")
    rbase = sid * RPT
    pltpu.sync_copy(zeros_hbm.at[pl.ds(rbase, RPT)], acc.at[pl.ds(rbase, RPT)])
    wbase = (cid * NS + sid) * CHUNKS
    plsc.subcore_barrier()

    def load_idx(j, q):
        pltpu.async_copy(src_hbm.at[pl.ds(wbase + j, 1)],
                         sidx.at[pl.ds(q, 1)], isems[q])
        pltpu.async_copy(dst_hbm.at[pl.ds(wbase + j, 1)],
                         didx.at[pl.ds(q, 1)], isems[q])

    def wait_idx(j, q):
        pltpu.make_async_copy(src_hbm.at[pl.ds(wbase + j, 1)],
                              sidx.at[pl.ds(q, 1)], isems[q]).wait()
        pltpu.make_async_copy(dst_hbm.at[pl.ds(wbase + j, 1)],
                              didx.at[pl.ds(q, 1)], isems[q]).wait()

    for q in range(3):
        load_idx(q, q)
    wait_idx(0, 0)
    pltpu.async_copy(table_hbm.at[sidx.at[0]], rows.at[0], gsem)

    def quad(i, carry):
        for u in range(4):
            j = i * 4 + u
            q = u
            b = u % 2
            nb = 1 - b
            pltpu.make_async_copy(table_hbm.at[sidx.at[q]], rows.at[b],
                                  gsem).wait()
            pltpu.async_copy(rows.at[b], acc.at[didx.at[q]], ssems[b],
                             add=True)
            @pl.when(j >= 1)
            def _():
                pltpu.make_async_copy(rows.at[nb], acc.at[didx.at[(q + 3) % 4]],
                                      ssems[nb]).wait()
            @pl.when(j + 1 < CHUNKS)
            def _():
                wait_idx(j + 1, (q + 1) % 4)
                pltpu.async_copy(table_hbm.at[sidx.at[(q + 1) % 4]],
                                 rows.at[nb], gsem)
            @pl.when(j + 3 < CHUNKS)
            def _():
                load_idx(j + 3, (q + 3) % 4)
        return carry

    lax.fori_loop(0, CHUNKS // 4, quad, 0)
    pltpu.make_async_copy(rows.at[1], acc.at[didx.at[3]], ssems[1]).wait()
    plsc.subcore_barrier()
    pltpu.sync_copy(acc.at[pl.ds(rbase, RPT)],
                    out_hbm.at[cid, pl.ds(rbase, RPT)])


@functools.partial(
    pl.kernel,
    out_type=jax.ShapeDtypeStruct((NC, NPAD, D), jnp.float32),
    mesh=_MESH,
    scratch_types=[
        pltpu.VMEM((CHUNKS, LANES), jnp.int32),
        pltpu.VMEM((LANES, D), jnp.float32),
        pltpu.VMEM_SHARED((NPAD, D), jnp.float32),
        pltpu.SemaphoreType.DMA,
    ],
)
def _deg_kernel(dst_hbm, ones_hbm, zeros_hbm, out_hbm, didx, ones_v, acc,
                ssem):
    cid = lax.axis_index("c")
    sid = lax.axis_index("s")
    rbase = sid * RPT
    pltpu.sync_copy(zeros_hbm.at[pl.ds(rbase, RPT)], acc.at[pl.ds(rbase, RPT)])
    pltpu.sync_copy(ones_hbm, ones_v)
    wbase = (cid * NS + sid) * CHUNKS
    pltpu.sync_copy(dst_hbm.at[pl.ds(wbase, CHUNKS)], didx)
    plsc.subcore_barrier()

    def fire(i, carry):
        pltpu.async_copy(ones_v, acc.at[didx.at[i]], ssem, add=True)
        return carry

    lax.fori_loop(0, CHUNKS, fire, 0)

    def drain(i, carry):
        pltpu.make_async_copy(ones_v, acc.at[didx.at[0]], ssem).wait()
        return carry

    lax.fori_loop(0, CHUNKS, drain, 0)
    plsc.subcore_barrier()
    pltpu.sync_copy(acc.at[pl.ds(rbase, RPT)],
                    out_hbm.at[cid, pl.ds(rbase, RPT)])


def _leaky(x):
    return jnp.where(x >= 0, x, 0.01 * x)


def _bn(h, g, be):
    mu = jnp.mean(h, axis=0, keepdims=True)
    var = jnp.mean((h - mu) ** 2, axis=0, keepdims=True)
    return g * (h - mu) * lax.rsqrt(var + EPS) + be


def _dinv_col(degp):
    deg = degp[0, :N, 0:1] + degp[1, :N, 0:1] + 1.0
    return lax.rsqrt(deg)


def _stage1_body(x_ref, w_ref, b_ref, g_ref, be_ref, wg_ref, degp_ref,
                 o_ref):
    h = jnp.dot(x_ref[...], w_ref[...],
                preferred_element_type=jnp.float32) + b_ref[...]
    h = _leaky(_bn(h, g_ref[...], be_ref[...]))
    h = h * lax.rsqrt(jnp.sum(h * h))
    dcol = _dinv_col(degp_ref[...])
    o_ref[...] = jnp.dot(h, wg_ref[...],
                         preferred_element_type=jnp.float32) * dcol


def _mid_body(parts_ref, xws_ref, degp_ref, w_ref, b_ref, g_ref, be_ref,
              o_ref):
    dcol = _dinv_col(degp_ref[...])
    parts = parts_ref[...]
    nb = dcol * (parts[0, :N, :] + parts[1, :N, :] + xws_ref[...]) + b_ref[...]
    h2 = _leaky(_bn(nb, g_ref[...], be_ref[...]))
    o_ref[...] = jnp.dot(h2, w_ref[...],
                         preferred_element_type=jnp.float32) * dcol


def _final_body(parts_ref, xws_ref, degp_ref, b_ref, g_ref, be_ref, o_ref):
    dcol = _dinv_col(degp_ref[...])
    parts = parts_ref[...]
    nb = dcol * (parts[0, :N, :] + parts[1, :N, :] + xws_ref[...]) + b_ref[...]
    h2 = _leaky(_bn(nb, g_ref[...], be_ref[...]))
    o_ref[...] = h2 * lax.rsqrt(jnp.sum(h2 * h2))


def _tc(body, *args):
    return pl.pallas_call(
        body, out_shape=jax.ShapeDtypeStruct((N, H), jnp.float32))(*args)


def kernel(x, edge_index, W_in, b_in, g1, be1, W_g, b_g, g2, be2):
    src = jnp.concatenate(
        [edge_index[0], jnp.arange(EPAD - E, dtype=jnp.int32) % N]
    ).reshape(NW * CHUNKS, LANES)
    dst = jnp.concatenate(
        [edge_index[1], jnp.full((EPAD - E,), N, jnp.int32)]
    ).reshape(NW * CHUNKS, LANES)

    zeros128 = jnp.zeros((NPAD, D), jnp.float32)
    ones_blk = jnp.ones((LANES, D), jnp.float32)

    brow = b_in.reshape(1, H)
    g1r = g1.reshape(1, H)
    be1r = be1.reshape(1, H)
    bgr = b_g.reshape(1, H)
    g2r = g2.reshape(1, H)
    be2r = be2.reshape(1, H)

    degp = _deg_kernel(dst, ones_blk, zeros128)

    xws = _tc(_stage1_body, x, W_in, brow, g1r, be1r, W_g, degp)
    for it in range(ITERS):
        parts = _seg_kernel(xws, src, dst, zeros128)
        if it < ITERS - 1:
            xws = _tc(_mid_body, parts, xws, degp, W_g, bgr, g2r, be2r)
        else:
            out = _tc(_final_body, parts, xws, degp, bgr, g2r, be2r)
    return out


# speedup vs baseline: 1.0921x; 1.0816x over previous
import functools

import jax
import jax.numpy as jnp
from jax import lax
from jax.experimental import pallas as pl
from jax.experimental.pallas import tpu as pltpu
from jax.experimental.pallas import tpu_sc as plsc

N = 10000
E = 320000
D = 128
H = 128
ITERS = 3
EPS = 1e-5

NC = 2
NS = 16
NW = NC * NS
LANES = 128

NPAD = 10240
RPT = NPAD // NS
EPAD = 327680
EPW = EPAD // NW
CHUNKS = EPW // LANES

_MESH = plsc.VectorSubcoreMesh(core_axis_name="c", subcore_axis_name="s")


NBUF = 2
GROUPS = CHUNKS // NBUF


@functools.partial(
    pl.kernel,
    out_type=jax.ShapeDtypeStruct((NC, NPAD, D), jnp.float32),
    mesh=_MESH,
    scratch_types=[
        pltpu.VMEM((4, LANES), jnp.int32),
        pltpu.VMEM((4, LANES), jnp.int32),
        pltpu.VMEM((NBUF, LANES, D), jnp.float32),
        pltpu.VMEM_SHARED((NPAD, D), jnp.float32),
        pltpu.SemaphoreType.DMA,
        pltpu.SemaphoreType.DMA,
        pltpu.SemaphoreType.DMA,
        [pltpu.SemaphoreType.DMA] * 4,
    ],
)
def _seg_kernel(table_hbm, src_hbm, dst_hbm, zeros_hbm, out_hbm,
                sidx, didx, rows, acc, gsem, ssem0, ssem1, isems):
    ssems = (ssem0, ssem1)
    cid = lax.axis_index("c")
    sid = lax.axis_index("s")
    rbase = sid * RPT
    pltpu.sync_copy(zeros_hbm.at[pl.ds(rbase, RPT)], acc.at[pl.ds(rbase, RPT)])
    wbase = (cid * NS + sid) * CHUNKS
    plsc.subcore_barrier()

    def load_idx(j, q):
        pltpu.async_copy(src_hbm.at[pl.ds(wbase + j, 1)],
                         sidx.at[pl.ds(q, 1)], isems[q])
        pltpu.async_copy(dst_hbm.at[pl.ds(wbase + j, 1)],
                         didx.at[pl.ds(q, 1)], isems[q])

    def wait_idx(j, q):
        pltpu.make_async_copy(src_hbm.at[pl.ds(wbase + j, 1)],
                              sidx.at[pl.ds(q, 1)], isems[q]).wait()
        pltpu.make_async_copy(dst_hbm.at[pl.ds(wbase + j, 1)],
                              didx.at[pl.ds(q, 1)], isems[q]).wait()

    for q in range(3):
        load_idx(q, q)
    wait_idx(0, 0)
    pltpu.async_copy(table_hbm.at[sidx.at[0]], rows.at[0], gsem)

    def quad(i, carry):
        for u in range(4):
            j = i * 4 + u
            q = u
            b = u % 2
            nb = 1 - b
            pltpu.make_async_copy(table_hbm.at[sidx.at[q]], rows.at[b],
                                  gsem).wait()
            pltpu.async_copy(rows.at[b], acc.at[didx.at[q]], ssems[b],
                             add=True)
            @pl.when(j >= 1)
            def _():
                pltpu.make_async_copy(rows.at[nb], acc.at[didx.at[(q + 3) % 4]],
                                      ssems[nb]).wait()
            @pl.when(j + 1 < CHUNKS)
            def _():
                wait_idx(j + 1, (q + 1) % 4)
                pltpu.async_copy(table_hbm.at[sidx.at[(q + 1) % 4]],
                                 rows.at[nb], gsem)
            @pl.when(j + 3 < CHUNKS)
            def _():
                load_idx(j + 3, (q + 3) % 4)
        return carry

    lax.fori_loop(0, CHUNKS // 4, quad, 0)
    pltpu.make_async_copy(rows.at[1], acc.at[didx.at[3]], ssems[1]).wait()
    plsc.subcore_barrier()
    pltpu.sync_copy(acc.at[pl.ds(rbase, RPT)],
                    out_hbm.at[cid, pl.ds(rbase, RPT)])


HCHUNK = 64


@functools.partial(
    pl.kernel,
    out_type=jax.ShapeDtypeStruct((NC, NPAD, D), jnp.float32),
    mesh=_MESH,
    scratch_types=[
        pltpu.VMEM((CHUNKS, LANES), jnp.int32),
        pltpu.VMEM((NPAD,), jnp.float32),
        pltpu.VMEM((NS, RPT), jnp.float32),
        pltpu.VMEM((16,), jnp.float32),
        pltpu.VMEM((HCHUNK, D), jnp.float32),
        pltpu.VMEM_SHARED((NS, NPAD), jnp.float32),
    ],
    compiler_params=pltpu.CompilerParams(needs_layout_passes=False),
)
def _deg_kernel(dst_hbm, out_hbm, didx, hist, red, svec, outbuf, shist):
    cid = lax.axis_index("c")
    sid = lax.axis_index("s")
    rbase = sid * RPT
    wbase = (cid * NS + sid) * CHUNKS
    pltpu.sync_copy(dst_hbm.at[pl.ds(wbase, CHUNKS)], didx)

    def zero(i, carry):
        hist[pl.ds(i * 16, 16)] = jnp.zeros((16,), jnp.float32)
        return carry

    lax.fori_loop(0, NPAD // 16, zero, 0)

    ones16 = jnp.ones((16,), jnp.float32)

    def hrow(r, carry):
        for c in range(LANES // 16):
            v = didx[r, pl.ds(c * 16, 16)]
            plsc.addupdate_scatter(hist, [v], ones16)
        return carry

    lax.fori_loop(0, CHUNKS, hrow, 0)

    pltpu.sync_copy(hist, shist.at[sid])
    plsc.subcore_barrier()
    for t in range(NS):
        pltpu.sync_copy(shist.at[t, pl.ds(rbase, RPT)], red.at[t])

    def rchunk(k, carry):
        for g in range(HCHUNK // 16):
            s = red[0, pl.ds(k * HCHUNK + g * 16, 16)]
            for t in range(1, NS):
                s = s + red[t, pl.ds(k * HCHUNK + g * 16, 16)]
            svec[...] = s
            for lane in range(16):
                splat = plsc.load_gather(
                    svec, [jnp.full((16,), lane, jnp.int32)])
                for c in range(D // 16):
                    outbuf[g * 16 + lane, pl.ds(c * 16, 16)] = splat
        pltpu.sync_copy(outbuf,
                        out_hbm.at[cid, pl.ds(rbase + k * HCHUNK, HCHUNK)])
        return carry

    lax.fori_loop(0, RPT // HCHUNK, rchunk, 0)


def _leaky(x):
    return jnp.where(x >= 0, x, 0.01 * x)


def _bn(h, g, be):
    mu = jnp.mean(h, axis=0, keepdims=True)
    var = jnp.mean((h - mu) ** 2, axis=0, keepdims=True)
    return g * (h - mu) * lax.rsqrt(var + EPS) + be


def _dinv_col(degp):
    deg = degp[0, :N, 0:1] + degp[1, :N, 0:1] + 1.0
    return lax.rsqrt(deg)


def _stage1_body(x_ref, w_ref, b_ref, g_ref, be_ref, wg_ref, degp_ref,
                 o_ref):
    h = jnp.dot(x_ref[...], w_ref[...],
                preferred_element_type=jnp.float32) + b_ref[...]
    h = _leaky(_bn(h, g_ref[...], be_ref[...]))
    h = h * lax.rsqrt(jnp.sum(h * h))
    dcol = _dinv_col(degp_ref[...])
    o_ref[...] = jnp.dot(h, wg_ref[...],
                         preferred_element_type=jnp.float32) * dcol


def _mid_body(parts_ref, xws_ref, degp_ref, w_ref, b_ref, g_ref, be_ref,
              o_ref):
    dcol = _dinv_col(degp_ref[...])
    parts = parts_ref[...]
    nb = dcol * (parts[0, :N, :] + parts[1, :N, :] + xws_ref[...]) + b_ref[...]
    h2 = _leaky(_bn(nb, g_ref[...], be_ref[...]))
    o_ref[...] = jnp.dot(h2, w_ref[...],
                         preferred_element_type=jnp.float32) * dcol


def _final_body(parts_ref, xws_ref, degp_ref, b_ref, g_ref, be_ref, o_ref):
    dcol = _dinv_col(degp_ref[...])
    parts = parts_ref[...]
    nb = dcol * (parts[0, :N, :] + parts[1, :N, :] + xws_ref[...]) + b_ref[...]
    h2 = _leaky(_bn(nb, g_ref[...], be_ref[...]))
    o_ref[...] = h2 * lax.rsqrt(jnp.sum(h2 * h2))


def _tc(body, *args):
    return pl.pallas_call(
        body, out_shape=jax.ShapeDtypeStruct((N, H), jnp.float32))(*args)


def kernel(x, edge_index, W_in, b_in, g1, be1, W_g, b_g, g2, be2):
    src = jnp.concatenate(
        [edge_index[0], jnp.arange(EPAD - E, dtype=jnp.int32) % N]
    ).reshape(NW * CHUNKS, LANES)
    dst = jnp.concatenate(
        [edge_index[1], jnp.full((EPAD - E,), N, jnp.int32)]
    ).reshape(NW * CHUNKS, LANES)

    zeros128 = jnp.zeros((NPAD, D), jnp.float32)

    brow = b_in.reshape(1, H)
    g1r = g1.reshape(1, H)
    be1r = be1.reshape(1, H)
    bgr = b_g.reshape(1, H)
    g2r = g2.reshape(1, H)
    be2r = be2.reshape(1, H)

    degp = _deg_kernel(dst)

    xws = _tc(_stage1_body, x, W_in, brow, g1r, be1r, W_g, degp)
    for it in range(ITERS):
        parts = _seg_kernel(xws, src, dst, zeros128)
        if it < ITERS - 1:
            xws = _tc(_mid_body, parts, xws, degp, W_g, bgr, g2r, be2r)
        else:
            out = _tc(_final_body, parts, xws, degp, bgr, g2r, be2r)
    return out


# speedup vs baseline: 1.0924x; 1.0003x over previous
import functools

import jax
import jax.numpy as jnp
from jax import lax
from jax.experimental import pallas as pl
from jax.experimental.pallas import tpu as pltpu
from jax.experimental.pallas import tpu_sc as plsc

N = 10000
E = 320000
D = 128
H = 128
ITERS = 3
EPS = 1e-5

NC = 2
NS = 16
NW = NC * NS
LANES = 128

NPAD = 10240
RPT = NPAD // NS
EPAD = 327680
EPW = EPAD // NW
CHUNKS = EPW // LANES

_MESH = plsc.VectorSubcoreMesh(core_axis_name="c", subcore_axis_name="s")


NBUF = 2


@functools.partial(
    pl.kernel,
    out_type=jax.ShapeDtypeStruct((NC, NPAD, D), jnp.float32),
    mesh=_MESH,
    scratch_types=[
        pltpu.VMEM((4, LANES), jnp.int32),
        pltpu.VMEM((4, LANES), jnp.int32),
        pltpu.VMEM((NBUF, LANES, D), jnp.float32),
        pltpu.VMEM_SHARED((NPAD, D), jnp.float32),
        pltpu.SemaphoreType.DMA,
        pltpu.SemaphoreType.DMA,
        pltpu.SemaphoreType.DMA,
        [pltpu.SemaphoreType.DMA] * 4,
    ],
)
def _seg_kernel(table_hbm, src_hbm, dst_hbm, zeros_hbm, out_hbm,
                sidx, didx, rows, acc, gsem, ssem0, ssem1, isems):
    ssems = (ssem0, ssem1)
    cid = lax.axis_index("c")
    sid = lax.axis_index("s")
    rbase = sid * RPT
    pltpu.sync_copy(zeros_hbm.at[pl.ds(rbase, RPT)], acc.at[pl.ds(rbase, RPT)])
    wbase = (cid * NS + sid) * CHUNKS
    plsc.subcore_barrier()

    def load_idx(j, q):
        pltpu.async_copy(src_hbm.at[pl.ds(wbase + j, 1)],
                         sidx.at[pl.ds(q, 1)], isems[q])
        pltpu.async_copy(dst_hbm.at[pl.ds(wbase + j, 1)],
                         didx.at[pl.ds(q, 1)], isems[q])

    def wait_idx(j, q):
        pltpu.make_async_copy(src_hbm.at[pl.ds(wbase + j, 1)],
                              sidx.at[pl.ds(q, 1)], isems[q]).wait()
        pltpu.make_async_copy(dst_hbm.at[pl.ds(wbase + j, 1)],
                              didx.at[pl.ds(q, 1)], isems[q]).wait()

    for q in range(3):
        load_idx(q, q)
    wait_idx(0, 0)
    pltpu.async_copy(table_hbm.at[sidx.at[0]], rows.at[0], gsem)

    def quad(i, carry):
        for u in range(4):
            j = i * 4 + u
            q = u
            b = u % 2
            nb = 1 - b
            pltpu.make_async_copy(table_hbm.at[sidx.at[q]], rows.at[b],
                                  gsem).wait()
            pltpu.async_copy(rows.at[b], acc.at[didx.at[q]], ssems[b],
                             add=True)
            @pl.when(j >= 1)
            def _():
                pltpu.make_async_copy(rows.at[nb], acc.at[didx.at[(q + 3) % 4]],
                                      ssems[nb]).wait()
            @pl.when(j + 1 < CHUNKS)
            def _():
                wait_idx(j + 1, (q + 1) % 4)
                pltpu.async_copy(table_hbm.at[sidx.at[(q + 1) % 4]],
                                 rows.at[nb], gsem)
            @pl.when(j + 3 < CHUNKS)
            def _():
                load_idx(j + 3, (q + 3) % 4)
        return carry

    lax.fori_loop(0, CHUNKS // 4, quad, 0)
    pltpu.make_async_copy(rows.at[1], acc.at[didx.at[3]], ssems[1]).wait()
    plsc.subcore_barrier()
    pltpu.sync_copy(acc.at[pl.ds(rbase, RPT)],
                    out_hbm.at[cid, pl.ds(rbase, RPT)])


HCHUNK = 64


@functools.partial(
    pl.kernel,
    out_type=jax.ShapeDtypeStruct((NC, NPAD, D), jnp.float32),
    mesh=_MESH,
    scratch_types=[
        pltpu.VMEM((CHUNKS, LANES), jnp.int32),
        pltpu.VMEM((NPAD,), jnp.float32),
        pltpu.VMEM((NS, RPT), jnp.float32),
        pltpu.VMEM((16,), jnp.float32),
        pltpu.VMEM((HCHUNK, D), jnp.float32),
        pltpu.VMEM_SHARED((NS, NPAD), jnp.float32),
    ],
    compiler_params=pltpu.CompilerParams(needs_layout_passes=False),
)
def _deg_kernel(dst_hbm, out_hbm, didx, hist, red, svec, outbuf, shist):
    cid = lax.axis_index("c")
    sid = lax.axis_index("s")
    rbase = sid * RPT
    wbase = (cid * NS + sid) * CHUNKS
    pltpu.sync_copy(dst_hbm.at[pl.ds(wbase, CHUNKS)], didx)

    def zero(i, carry):
        hist[pl.ds(i * 16, 16)] = jnp.zeros((16,), jnp.float32)
        return carry

    lax.fori_loop(0, NPAD // 16, zero, 0)

    ones16 = jnp.ones((16,), jnp.float32)

    def hrow(r, carry):
        for c in range(LANES // 16):
            v = didx[r, pl.ds(c * 16, 16)]
            plsc.addupdate_scatter(hist, [v], ones16)
        return carry

    lax.fori_loop(0, CHUNKS, hrow, 0)

    pltpu.sync_copy(hist, shist.at[sid])
    plsc.subcore_barrier()
    for t in range(NS):
        pltpu.sync_copy(shist.at[t, pl.ds(rbase, RPT)], red.at[t])

    def rchunk(k, carry):
        for g in range(HCHUNK // 16):
            s = red[0, pl.ds(k * HCHUNK + g * 16, 16)]
            for t in range(1, NS):
                s = s + red[t, pl.ds(k * HCHUNK + g * 16, 16)]
            svec[...] = s
            for lane in range(16):
                splat = plsc.load_gather(
                    svec, [jnp.full((16,), lane, jnp.int32)])
                for c in range(D // 16):
                    outbuf[g * 16 + lane, pl.ds(c * 16, 16)] = splat
        pltpu.sync_copy(outbuf,
                        out_hbm.at[cid, pl.ds(rbase + k * HCHUNK, HCHUNK)])
        return carry

    lax.fori_loop(0, RPT // HCHUNK, rchunk, 0)


def _leaky(x):
    return jnp.where(x >= 0, x, 0.01 * x)


def _bn(h, g, be):
    mu = jnp.mean(h, axis=0, keepdims=True)
    var = jnp.mean((h - mu) ** 2, axis=0, keepdims=True)
    return g * (h - mu) * lax.rsqrt(var + EPS) + be


def _dinv_col(degp):
    deg = degp[0, :N, 0:1] + degp[1, :N, 0:1] + 1.0
    return lax.rsqrt(deg)


def _stage1_body(x_ref, w_ref, b_ref, g_ref, be_ref, wg_ref, degp_ref,
                 o_ref):
    h = jnp.dot(x_ref[...], w_ref[...],
                preferred_element_type=jnp.float32) + b_ref[...]
    h = _leaky(_bn(h, g_ref[...], be_ref[...]))
    h = h * lax.rsqrt(jnp.sum(h * h))
    dcol = _dinv_col(degp_ref[...])
    o_ref[...] = jnp.dot(h, wg_ref[...],
                         preferred_element_type=jnp.float32) * dcol


def _mid_body(parts_ref, xws_ref, degp_ref, w_ref, b_ref, g_ref, be_ref,
              o_ref):
    dcol = _dinv_col(degp_ref[...])
    parts = parts_ref[...]
    nb = dcol * (parts[0, :N, :] + parts[1, :N, :] + xws_ref[...]) + b_ref[...]
    h2 = _leaky(_bn(nb, g_ref[...], be_ref[...]))
    o_ref[...] = jnp.dot(h2, w_ref[...],
                         preferred_element_type=jnp.float32) * dcol


def _final_body(parts_ref, xws_ref, degp_ref, b_ref, g_ref, be_ref, o_ref):
    dcol = _dinv_col(degp_ref[...])
    parts = parts_ref[...]
    nb = dcol * (parts[0, :N, :] + parts[1, :N, :] + xws_ref[...]) + b_ref[...]
    h2 = _leaky(_bn(nb, g_ref[...], be_ref[...]))
    o_ref[...] = h2 * lax.rsqrt(jnp.sum(h2 * h2))


def _tc(body, *args):
    return pl.pallas_call(
        body, out_shape=jax.ShapeDtypeStruct((N, H), jnp.float32))(*args)


def kernel(x, edge_index, W_in, b_in, g1, be1, W_g, b_g, g2, be2):
    src = jnp.concatenate(
        [edge_index[0], jnp.arange(EPAD - E, dtype=jnp.int32) % N]
    ).reshape(NW * CHUNKS, LANES)
    dst = jnp.concatenate(
        [edge_index[1], jnp.full((EPAD - E,), N, jnp.int32)]
    ).reshape(NW * CHUNKS, LANES)

    zeros128 = jnp.zeros((NPAD, D), jnp.float32)

    brow = b_in.reshape(1, H)
    g1r = g1.reshape(1, H)
    be1r = be1.reshape(1, H)
    bgr = b_g.reshape(1, H)
    g2r = g2.reshape(1, H)
    be2r = be2.reshape(1, H)

    degp = _deg_kernel(dst)

    xws = _tc(_stage1_body, x, W_in, brow, g1r, be1r, W_g, degp)
    for it in range(ITERS):
        parts = _seg_kernel(xws, src, dst, zeros128)
        if it < ITERS - 1:
            xws = _tc(_mid_body, parts, xws, degp, W_g, bgr, g2r, be2r)
        else:
            out = _tc(_final_body, parts, xws, degp, bgr, g2r, be2r)
    return out
